# baseline (speedup 1.0000x reference)
import functools
import jax
import jax.numpy as jnp
from jax import lax
from jax.experimental import pallas as pl
from jax.experimental.pallas import tpu as pltpu
from jax.experimental.pallas import tpu_sc as plsc

TOPK = 2
NG = 4
GSIZE = 4
SCALE = 2.5
NEG = -1e30

BM = 128
BI = 256


def _router_body(bias_ref, x_ref, gw_ref, comb_ref):
    x = x_ref[...]
    gw = gw_ref[...]
    logits = jax.lax.dot_general(
        x, gw, (((1,), (1,)), ((), ())), preferred_element_type=jnp.float32
    )
    scores = jax.nn.sigmoid(logits)
    swb = scores + bias_ref[...]
    bm = x.shape[0]
    ne = gw.shape[0]
    lane = jax.lax.broadcasted_iota(jnp.int32, (bm, ne), 1)
    grp = lane // GSIZE

    gs = []
    for g in range(NG):
        v = jnp.where(grp == g, swb, NEG)
        m1 = jnp.max(v, axis=1, keepdims=True)
        i1 = jnp.min(jnp.where(v == m1, lane, 999), axis=1, keepdims=True)
        v2 = jnp.where(lane == i1, NEG, v)
        m2 = jnp.max(v2, axis=1, keepdims=True)
        gs.append(m1 + m2)
    gsm = jnp.concatenate(gs, axis=1)

    lane4 = jax.lax.broadcasted_iota(jnp.int32, (bm, NG), 1)
    g1v = jnp.max(gsm, axis=1, keepdims=True)
    g1 = jnp.min(jnp.where(gsm == g1v, lane4, 999), axis=1, keepdims=True)
    gsm2 = jnp.where(lane4 == g1, NEG, gsm)
    g2v = jnp.max(gsm2, axis=1, keepdims=True)
    g2 = jnp.min(jnp.where(gsm2 == g2v, lane4, 999), axis=1, keepdims=True)
    gmask = (grp == g1) | (grp == g2)

    swbm = jnp.where(gmask, swb, 0.0)
    m1 = jnp.max(swbm, axis=1, keepdims=True)
    e1 = jnp.min(jnp.where(swbm == m1, lane, 999), axis=1, keepdims=True)
    swbm2 = jnp.where(lane == e1, NEG, swbm)
    m2 = jnp.max(swbm2, axis=1, keepdims=True)
    e2 = jnp.min(jnp.where(swbm2 == m2, lane, 999), axis=1, keepdims=True)
    sel = (lane == e1) | (lane == e2)

    sm = jnp.where(sel, scores, 0.0)
    ssum = jnp.sum(sm, axis=1, keepdims=True) + 1e-20
    comb_ref[...] = sm / ssum * SCALE


def _router(x, gate_weight, bias):
    T, H = x.shape
    E = gate_weight.shape[0]
    BMR = 256
    return pl.pallas_call(
        _router_body,
        grid=(T // BMR,),
        in_specs=[
            pl.BlockSpec((1, E), lambda i: (0, 0)),
            pl.BlockSpec((BMR, H), lambda i: (i, 0)),
            pl.BlockSpec((E, H), lambda i: (0, 0)),
        ],
        out_specs=pl.BlockSpec((BMR, E), lambda i: (i, 0)),
        out_shape=jax.ShapeDtypeStruct((T, E), jnp.float32),
    )(bias.reshape(1, E), x, gate_weight)


def _mlp_body(x_ref, wg_ref, wu_ref, wd_ref, out_ref):
    @pl.when(pl.program_id(1) == 0)
    def _():
        out_ref[...] = jnp.zeros_like(out_ref)

    x = x_ref[...]
    g = jnp.dot(x, wg_ref[...], preferred_element_type=jnp.float32)
    u = jnp.dot(x, wu_ref[...], preferred_element_type=jnp.float32)
    h = (g * jax.nn.sigmoid(g) * u).astype(jnp.bfloat16)
    out_ref[...] += jnp.dot(h, wd_ref[...], preferred_element_type=jnp.float32)


def _shared_mlp(x, wg, wu, wd):
    T, H = x.shape
    SI = wg.shape[1]
    BMS, BIS = 256, 256
    return pl.pallas_call(
        _mlp_body,
        grid=(T // BMS, SI // BIS),
        in_specs=[
            pl.BlockSpec((BMS, H), lambda m, i: (m, 0)),
            pl.BlockSpec((H, BIS), lambda m, i: (0, i)),
            pl.BlockSpec((H, BIS), lambda m, i: (0, i)),
            pl.BlockSpec((BIS, H), lambda m, i: (i, 0)),
        ],
        out_specs=pl.BlockSpec((BMS, H), lambda m, i: (m, 0)),
        out_shape=jax.ShapeDtypeStruct((T, H), jnp.float32),
    )(
        x.astype(jnp.bfloat16),
        wg.astype(jnp.bfloat16),
        wu.astype(jnp.bfloat16),
        wd.astype(jnp.bfloat16),
    )


def _dispatch_plan(combine, P):
    T, E = combine.shape
    mask = combine > 0.0
    mi = mask.astype(jnp.int32)
    cum = jnp.cumsum(mi, axis=0) - mi
    counts = jnp.sum(mi, axis=0)
    padded = ((counts + BM - 1) // BM) * BM
    ends = jnp.cumsum(padded)
    offs = ends - padded
    slot = offs[None, :] + cum
    slot_full = jnp.where(mask, slot, P - 1)

    tok = jnp.broadcast_to(jnp.arange(T, dtype=jnp.int32)[:, None], (T, E))
    tid_sorted = jnp.zeros((P,), jnp.int32).at[slot_full.reshape(-1)].set(
        tok.reshape(-1)
    )
    w_sorted = jnp.zeros((P,), jnp.float32).at[slot_full.reshape(-1)].set(
        jnp.where(mask, combine, 0.0).reshape(-1)
    )

    tile_starts = jnp.arange(P // BM, dtype=jnp.int32) * BM
    tile_expert = jnp.minimum(
        jnp.searchsorted(ends, tile_starts, side="right").astype(jnp.int32),
        E - 1,
    )

    sel_slot = jnp.where(mask, slot, -1)
    top2 = jax.lax.top_k(sel_slot, TOPK)[0]
    slot0 = jnp.where(top2[:, 0] >= 0, top2[:, 0], P - 1).astype(jnp.int32)
    slot1 = jnp.where(top2[:, 1] >= 0, top2[:, 1], P - 1).astype(jnp.int32)
    return tid_sorted, w_sorted, tile_expert, slot0, slot1


def _sc_gather(x, tid_sorted, P):
    T, H = x.shape
    info = plsc.get_sparse_core_info()
    NW = info.num_cores * info.num_subcores
    per_w = P // NW
    CH = 48
    n_chunks = per_w // CH
    mesh = plsc.VectorSubcoreMesh(core_axis_name="c", subcore_axis_name="s")

    @functools.partial(
        pl.kernel,
        mesh=mesh,
        out_type=jax.ShapeDtypeStruct((P, H), jnp.float32),
        scratch_types=[
            pltpu.VMEM((CH,), jnp.int32),
            pltpu.VMEM((CH, H), jnp.float32),
            pltpu.SemaphoreType.DMA,
        ],
    )
    def gather_k(x_hbm, tid_hbm, out_hbm, idx_v, rows_v, sem):
        wid = lax.axis_index("s") * info.num_cores + lax.axis_index("c")
        base = wid * per_w

        def chunk(c, carry):
            off = base + c * CH
            pltpu.sync_copy(tid_hbm.at[pl.ds(off, CH)], idx_v)
            pltpu.async_copy(x_hbm.at[idx_v], rows_v, sem).wait()
            pltpu.sync_copy(rows_v, out_hbm.at[pl.ds(off, CH)])
            return carry

        lax.fori_loop(0, n_chunks, chunk, 0)

    return gather_k(x, tid_sorted)


def _gemm_body(te_ref, xg_ref, ws_ref, wg_ref, wu_ref, wd_ref, out_ref):
    @pl.when(pl.program_id(1) == 0)
    def _():
        out_ref[...] = jnp.zeros_like(out_ref)

    x = xg_ref[...]
    g = jnp.dot(x, wg_ref[0], preferred_element_type=jnp.float32)
    u = jnp.dot(x, wu_ref[0], preferred_element_type=jnp.float32)
    h = (g * jax.nn.sigmoid(g) * u * ws_ref[...]).astype(jnp.bfloat16)
    out_ref[...] += jnp.dot(h, wd_ref[0], preferred_element_type=jnp.float32)


def _grouped_gemm(xg, w_sorted, tile_expert, w_gate, w_up, w_down, P):
    H = xg.shape[1]
    E, _, I = w_gate.shape
    grid = (P // BM, I // BI)
    return pl.pallas_call(
        _gemm_body,
        grid_spec=pltpu.PrefetchScalarGridSpec(
            num_scalar_prefetch=1,
            grid=grid,
            in_specs=[
                pl.BlockSpec((BM, H), lambda m, i, te: (m, 0)),
                pl.BlockSpec((BM, 1), lambda m, i, te: (m, 0)),
                pl.BlockSpec((1, H, BI), lambda m, i, te: (te[m], 0, i)),
                pl.BlockSpec((1, H, BI), lambda m, i, te: (te[m], 0, i)),
                pl.BlockSpec((1, BI, H), lambda m, i, te: (te[m], i, 0)),
            ],
            out_specs=pl.BlockSpec((BM, H), lambda m, i, te: (m, 0)),
        ),
        out_shape=jax.ShapeDtypeStruct((P, H), jnp.float32),
    )(
        tile_expert,
        xg.astype(jnp.bfloat16),
        w_sorted.reshape(P, 1),
        w_gate.astype(jnp.bfloat16),
        w_up.astype(jnp.bfloat16),
        w_down.astype(jnp.bfloat16),
    )


def _sc_combine(yg, shared, slot0, slot1):
    T, H = shared.shape
    info = plsc.get_sparse_core_info()
    NW = info.num_cores * info.num_subcores
    per_w = T // NW
    CH = 16
    n_chunks = per_w // CH
    mesh = plsc.VectorSubcoreMesh(core_axis_name="c", subcore_axis_name="s")

    @functools.partial(
        pl.kernel,
        mesh=mesh,
        out_type=jax.ShapeDtypeStruct((T, H), jnp.float32),
        scratch_types=[
            pltpu.VMEM((CH,), jnp.int32),
            pltpu.VMEM((CH,), jnp.int32),
            pltpu.VMEM((CH, H), jnp.float32),
            pltpu.VMEM((CH, H), jnp.float32),
            pltpu.VMEM((CH, H), jnp.float32),
            pltpu.SemaphoreType.DMA,
        ],
    )
    def combine_k(yg_hbm, sh_hbm, s0_hbm, s1_hbm, out_hbm,
                  i0_v, i1_v, a_v, b_v, s_v, sem):
        wid = lax.axis_index("s") * info.num_cores + lax.axis_index("c")
        base = wid * per_w

        def chunk(c, carry):
            off = base + c * CH
            pltpu.sync_copy(s0_hbm.at[pl.ds(off, CH)], i0_v)
            pltpu.sync_copy(s1_hbm.at[pl.ds(off, CH)], i1_v)
            pltpu.async_copy(yg_hbm.at[i0_v], a_v, sem).wait()
            pltpu.async_copy(yg_hbm.at[i1_v], b_v, sem).wait()
            pltpu.sync_copy(sh_hbm.at[pl.ds(off, CH)], s_v)

            def add_row(i, carry2):
                def add_col(c2, carry3):
                    sl = pl.ds(c2 * 16, 16)
                    s_v[i, sl] = s_v[i, sl] + a_v[i, sl] + b_v[i, sl]
                    return carry3

                lax.fori_loop(0, H // 16, add_col, 0)
                return carry2

            lax.fori_loop(0, CH, add_row, 0)
            pltpu.sync_copy(s_v, out_hbm.at[pl.ds(off, CH)])
            return carry

        lax.fori_loop(0, n_chunks, chunk, 0)

    return combine_k(yg, shared, slot0, slot1)


def kernel(hidden_states, gate_weight, e_score_correction_bias, w_gate, w_up,
           w_down, shared_w_gate, shared_w_up, shared_w_down):
    x = hidden_states
    T, H = x.shape
    E = gate_weight.shape[0]
    P = T * TOPK + E * BM

    combine = _router(x, gate_weight, e_score_correction_bias)
    shared = _shared_mlp(x, shared_w_gate, shared_w_up, shared_w_down)
    tid_sorted, w_sorted, tile_expert, slot0, slot1 = _dispatch_plan(combine, P)
    xg = _sc_gather(x, tid_sorted, P)
    yg = _grouped_gemm(xg, w_sorted, tile_expert, w_gate, w_up, w_down, P)
    return _sc_combine(yg, shared, slot0, slot1)


# speedup vs baseline: 1.5229x; 1.5229x over previous
import functools
import jax
import jax.numpy as jnp
from jax import lax
from jax.experimental import pallas as pl
from jax.experimental.pallas import tpu as pltpu
from jax.experimental.pallas import tpu_sc as plsc

TOPK = 2
NG = 4
GSIZE = 4
SCALE = 2.5
NEG = -1e30

BM = 128
BI = 256


def _router_body(bias_ref, x_ref, gw_ref, comb_ref):
    x = x_ref[...]
    gw = gw_ref[...]
    logits = jax.lax.dot_general(
        x, gw, (((1,), (1,)), ((), ())), preferred_element_type=jnp.float32
    )
    scores = jax.nn.sigmoid(logits)
    swb = scores + bias_ref[...]
    bm = x.shape[0]
    ne = gw.shape[0]
    lane = jax.lax.broadcasted_iota(jnp.int32, (bm, ne), 1)
    grp = lane // GSIZE

    gs = []
    for g in range(NG):
        v = jnp.where(grp == g, swb, NEG)
        m1 = jnp.max(v, axis=1, keepdims=True)
        i1 = jnp.min(jnp.where(v == m1, lane, 999), axis=1, keepdims=True)
        v2 = jnp.where(lane == i1, NEG, v)
        m2 = jnp.max(v2, axis=1, keepdims=True)
        gs.append(m1 + m2)
    gsm = jnp.concatenate(gs, axis=1)

    lane4 = jax.lax.broadcasted_iota(jnp.int32, (bm, NG), 1)
    g1v = jnp.max(gsm, axis=1, keepdims=True)
    g1 = jnp.min(jnp.where(gsm == g1v, lane4, 999), axis=1, keepdims=True)
    gsm2 = jnp.where(lane4 == g1, NEG, gsm)
    g2v = jnp.max(gsm2, axis=1, keepdims=True)
    g2 = jnp.min(jnp.where(gsm2 == g2v, lane4, 999), axis=1, keepdims=True)
    gmask = (grp == g1) | (grp == g2)

    swbm = jnp.where(gmask, swb, 0.0)
    m1 = jnp.max(swbm, axis=1, keepdims=True)
    e1 = jnp.min(jnp.where(swbm == m1, lane, 999), axis=1, keepdims=True)
    swbm2 = jnp.where(lane == e1, NEG, swbm)
    m2 = jnp.max(swbm2, axis=1, keepdims=True)
    e2 = jnp.min(jnp.where(swbm2 == m2, lane, 999), axis=1, keepdims=True)
    sel = (lane == e1) | (lane == e2)

    sm = jnp.where(sel, scores, 0.0)
    ssum = jnp.sum(sm, axis=1, keepdims=True) + 1e-20
    comb_ref[...] = sm / ssum * SCALE


def _router(x, gate_weight, bias):
    T, H = x.shape
    E = gate_weight.shape[0]
    BMR = 256
    return pl.pallas_call(
        _router_body,
        grid=(T // BMR,),
        in_specs=[
            pl.BlockSpec((1, E), lambda i: (0, 0)),
            pl.BlockSpec((BMR, H), lambda i: (i, 0)),
            pl.BlockSpec((E, H), lambda i: (0, 0)),
        ],
        out_specs=pl.BlockSpec((BMR, E), lambda i: (i, 0)),
        out_shape=jax.ShapeDtypeStruct((T, E), jnp.float32),
    )(bias.reshape(1, E), x, gate_weight)


def _mlp_body(x_ref, wg_ref, wu_ref, wd_ref, out_ref):
    @pl.when(pl.program_id(1) == 0)
    def _():
        out_ref[...] = jnp.zeros_like(out_ref)

    x = x_ref[...]
    g = jnp.dot(x, wg_ref[...], preferred_element_type=jnp.float32)
    u = jnp.dot(x, wu_ref[...], preferred_element_type=jnp.float32)
    h = g * jax.nn.sigmoid(g) * u
    out_ref[...] += jnp.dot(h, wd_ref[...], preferred_element_type=jnp.float32)


def _shared_mlp(x, wg, wu, wd):
    T, H = x.shape
    SI = wg.shape[1]
    BMS, BIS = 512, 512
    return pl.pallas_call(
        _mlp_body,
        grid=(T // BMS, SI // BIS),
        in_specs=[
            pl.BlockSpec((BMS, H), lambda m, i: (m, 0)),
            pl.BlockSpec((H, BIS), lambda m, i: (0, i)),
            pl.BlockSpec((H, BIS), lambda m, i: (0, i)),
            pl.BlockSpec((BIS, H), lambda m, i: (i, 0)),
        ],
        out_specs=pl.BlockSpec((BMS, H), lambda m, i: (m, 0)),
        out_shape=jax.ShapeDtypeStruct((T, H), jnp.float32),
    )(x, wg, wu, wd)


def _dispatch_plan(combine, P):
    T, E = combine.shape
    mask = combine > 0.0
    mi = mask.astype(jnp.int32)
    cum = jnp.cumsum(mi, axis=0) - mi
    counts = jnp.sum(mi, axis=0)
    padded = ((counts + BM - 1) // BM) * BM
    ends = jnp.cumsum(padded)
    offs = ends - padded
    slot = offs[None, :] + cum
    slot_full = jnp.where(mask, slot, P - 1)

    tok = jnp.broadcast_to(jnp.arange(T, dtype=jnp.int32)[:, None], (T, E))
    tid_sorted = jnp.zeros((P,), jnp.int32).at[slot_full.reshape(-1)].set(
        tok.reshape(-1)
    )
    w_sorted = jnp.zeros((P,), jnp.float32).at[slot_full.reshape(-1)].set(
        jnp.where(mask, combine, 0.0).reshape(-1)
    )

    tile_starts = jnp.arange(P // BM, dtype=jnp.int32) * BM
    tile_expert = jnp.minimum(
        jnp.searchsorted(ends, tile_starts, side="right").astype(jnp.int32),
        E - 1,
    )

    sel_slot = jnp.where(mask, slot, -1)
    top2 = jax.lax.top_k(sel_slot, TOPK)[0]
    slot0 = jnp.where(top2[:, 0] >= 0, top2[:, 0], P - 1).astype(jnp.int32)
    slot1 = jnp.where(top2[:, 1] >= 0, top2[:, 1], P - 1).astype(jnp.int32)
    return tid_sorted, w_sorted, tile_expert, slot0, slot1


def _sc_gather(x, tid_sorted, P):
    T, H = x.shape
    info = plsc.get_sparse_core_info()
    NW = info.num_cores * info.num_subcores
    per_w = P // NW
    CH = 48
    n_chunks = per_w // CH
    mesh = plsc.VectorSubcoreMesh(core_axis_name="c", subcore_axis_name="s")

    @functools.partial(
        pl.kernel,
        mesh=mesh,
        out_type=jax.ShapeDtypeStruct((P, H), jnp.float32),
        scratch_types=[
            pltpu.VMEM((CH,), jnp.int32),
            pltpu.VMEM((CH, H), jnp.float32),
            pltpu.SemaphoreType.DMA,
        ],
    )
    def gather_k(x_hbm, tid_hbm, out_hbm, idx_v, rows_v, sem):
        wid = lax.axis_index("s") * info.num_cores + lax.axis_index("c")
        base = wid * per_w

        def chunk(c, carry):
            off = base + c * CH
            pltpu.sync_copy(tid_hbm.at[pl.ds(off, CH)], idx_v)
            pltpu.async_copy(x_hbm.at[idx_v], rows_v, sem).wait()
            pltpu.sync_copy(rows_v, out_hbm.at[pl.ds(off, CH)])
            return carry

        lax.fori_loop(0, n_chunks, chunk, 0)

    return gather_k(x, tid_sorted)


def _gemm_body(te_ref, xg_ref, ws_ref, wg_ref, wu_ref, wd_ref, out_ref):
    x = xg_ref[...]
    g = jnp.dot(x, wg_ref[0], preferred_element_type=jnp.float32)
    u = jnp.dot(x, wu_ref[0], preferred_element_type=jnp.float32)
    h = g * jax.nn.sigmoid(g) * u * ws_ref[...]
    out_ref[...] = jnp.dot(h, wd_ref[0], preferred_element_type=jnp.float32)


def _grouped_gemm(xg, w_sorted, tile_expert, w_gate, w_up, w_down, P):
    H = xg.shape[1]
    E, _, I = w_gate.shape
    return pl.pallas_call(
        _gemm_body,
        grid_spec=pltpu.PrefetchScalarGridSpec(
            num_scalar_prefetch=1,
            grid=(P // BM,),
            in_specs=[
                pl.BlockSpec((BM, H), lambda m, te: (m, 0)),
                pl.BlockSpec((BM, 1), lambda m, te: (m, 0)),
                pl.BlockSpec((1, H, I), lambda m, te: (te[m], 0, 0)),
                pl.BlockSpec((1, H, I), lambda m, te: (te[m], 0, 0)),
                pl.BlockSpec((1, I, H), lambda m, te: (te[m], 0, 0)),
            ],
            out_specs=pl.BlockSpec((BM, H), lambda m, te: (m, 0)),
        ),
        out_shape=jax.ShapeDtypeStruct((P, H), jnp.float32),
    )(tile_expert, xg, w_sorted.reshape(P, 1), w_gate, w_up, w_down)


def _sc_combine(yg, shared, slot0, slot1):
    T, H = shared.shape
    info = plsc.get_sparse_core_info()
    NW = info.num_cores * info.num_subcores
    per_w = T // NW
    CH = 16
    n_chunks = per_w // CH
    mesh = plsc.VectorSubcoreMesh(core_axis_name="c", subcore_axis_name="s")

    @functools.partial(
        pl.kernel,
        mesh=mesh,
        out_type=jax.ShapeDtypeStruct((T, H), jnp.float32),
        scratch_types=[
            pltpu.VMEM((CH,), jnp.int32),
            pltpu.VMEM((CH,), jnp.int32),
            pltpu.VMEM((CH, H), jnp.float32),
            pltpu.VMEM((CH, H), jnp.float32),
            pltpu.VMEM((CH, H), jnp.float32),
            pltpu.SemaphoreType.DMA,
        ],
    )
    def combine_k(yg_hbm, sh_hbm, s0_hbm, s1_hbm, out_hbm,
                  i0_v, i1_v, a_v, b_v, s_v, sem):
        wid = lax.axis_index("s") * info.num_cores + lax.axis_index("c")
        base = wid * per_w

        def chunk(c, carry):
            off = base + c * CH
            pltpu.sync_copy(s0_hbm.at[pl.ds(off, CH)], i0_v)
            pltpu.sync_copy(s1_hbm.at[pl.ds(off, CH)], i1_v)
            pltpu.async_copy(yg_hbm.at[i0_v], a_v, sem).wait()
            pltpu.async_copy(yg_hbm.at[i1_v], b_v, sem).wait()
            pltpu.sync_copy(sh_hbm.at[pl.ds(off, CH)], s_v)

            def add_row(i, carry2):
                def add_col(c2, carry3):
                    sl = pl.ds(c2 * 16, 16)
                    s_v[i, sl] = s_v[i, sl] + a_v[i, sl] + b_v[i, sl]
                    return carry3

                lax.fori_loop(0, H // 16, add_col, 0)
                return carry2

            lax.fori_loop(0, CH, add_row, 0)
            pltpu.sync_copy(s_v, out_hbm.at[pl.ds(off, CH)])
            return carry

        lax.fori_loop(0, n_chunks, chunk, 0)

    return combine_k(yg, shared, slot0, slot1)


def kernel(hidden_states, gate_weight, e_score_correction_bias, w_gate, w_up,
           w_down, shared_w_gate, shared_w_up, shared_w_down):
    x = hidden_states
    T, H = x.shape
    E = gate_weight.shape[0]
    P = T * TOPK + E * BM

    combine = _router(x, gate_weight, e_score_correction_bias)
    shared = _shared_mlp(x, shared_w_gate, shared_w_up, shared_w_down)
    tid_sorted, w_sorted, tile_expert, slot0, slot1 = _dispatch_plan(combine, P)
    xg = _sc_gather(x, tid_sorted, P)
    yg = _grouped_gemm(xg, w_sorted, tile_expert, w_gate, w_up, w_down, P)
    return _sc_combine(yg, shared, slot0, slot1)
